# baseline (speedup 1.0000x reference)
import functools

import jax
import jax.numpy as jnp
from jax import lax
from jax.experimental import pallas as pl
from jax.experimental.pallas import tpu as pltpu
from jax.experimental.pallas import tpu_sc as plsc


def _transpose_table(tt):
    D, V = tt.shape
    vb = 16384
    grid = (V + vb - 1) // vb

    def body(in_ref, out_ref):
        eye = (jax.lax.broadcasted_iota(jnp.int32, (D, D), 0)
               == jax.lax.broadcasted_iota(jnp.int32, (D, D), 1)
               ).astype(jnp.float32)
        out_ref[...] = jax.lax.dot_general(
            in_ref[...], eye, (((0,), (0,)), ((), ())),
            preferred_element_type=jnp.float32)

    return pl.pallas_call(
        body,
        grid=(grid,),
        in_specs=[pl.BlockSpec((D, vb), lambda i: (0, i))],
        out_specs=pl.BlockSpec((vb, D), lambda i: (i, 0)),
        out_shape=jax.ShapeDtypeStruct((V, D), jnp.float32),
    )(tt)


def _make_gather(B, D, num_workers, chunk, nbuf):
    b_per_w = B // num_workers
    nstep = b_per_w // chunk
    mesh = plsc.VectorSubcoreMesh(core_axis_name="c", subcore_axis_name="s")

    @functools.partial(
        pl.kernel,
        mesh=mesh,
        compiler_params=pltpu.CompilerParams(use_tc_tiling_on_sc=False),
        out_type=jax.ShapeDtypeStruct((B, D), jnp.float32),
        scratch_types=[
            pltpu.VMEM((b_per_w,), jnp.int32),
            pltpu.VMEM((nbuf, chunk, D), jnp.float32),
            pltpu.SemaphoreType.DMA((nbuf,)),
            pltpu.SemaphoreType.DMA((nbuf,)),
        ],
    )
    def gather(idx_hbm, table_hbm, out_hbm, idx_v, rows_v, gsem, wsem):
        wid = lax.axis_index("s") * 2 + lax.axis_index("c")
        base = wid * b_per_w
        pltpu.sync_copy(idx_hbm.at[pl.ds(base, b_per_w)], idx_v)

        def start_gather(g):
            b = g % nbuf
            return pltpu.async_copy(
                table_hbm.at[idx_v.at[pl.ds(g * chunk, chunk)]],
                rows_v.at[b],
                gsem.at[b],
            )

        def start_write(g):
            b = g % nbuf
            return pltpu.async_copy(
                rows_v.at[b],
                out_hbm.at[pl.ds(base + g * chunk, chunk)],
                wsem.at[b],
            )

        gh = [None] * nstep
        wh = [None] * nstep
        for g in range(min(nbuf, nstep)):
            gh[g] = start_gather(g)
        for g in range(nstep):
            gh[g].wait()
            wh[g] = start_write(g)
            nxt = g + nbuf
            if nxt < nstep:
                wh[g].wait()
                gh[nxt] = start_gather(nxt)
        for g in range(max(0, nstep - nbuf), nstep):
            wh[g].wait()

    return gather


def kernel(x, table):
    B = x.size
    D = table.shape[1]
    t_rm = _transpose_table(table.T)
    idx = x.reshape(B).astype(jnp.int32)
    out = _make_gather(B, D, 32, 1280, 4)(idx, t_rm)
    return out.reshape(*x.shape, D)


# speedup vs baseline: 1.3831x; 1.3831x over previous
import functools

import jax
import jax.numpy as jnp
from jax import lax
from jax.experimental import pallas as pl
from jax.experimental.pallas import tpu as pltpu
from jax.experimental.pallas import tpu_sc as plsc


def _iota16():
    return lax.iota(jnp.int32, 16)


def _make_gather_native(B4096, J200, V, D):
    mesh = plsc.VectorSubcoreMesh(core_axis_name="c", subcore_axis_name="s")
    n_chunks = J200 // 8

    @functools.partial(
        pl.kernel,
        mesh=mesh,
        compiler_params=pltpu.CompilerParams(
            use_tc_tiling_on_sc=False, needs_layout_passes=False),
        out_type=jax.ShapeDtypeStruct((J200, 2, 32, 8, 128), jnp.float32),
        scratch_types=[
            pltpu.VMEM((128, J200), jnp.int32),
            pltpu.VMEM((n_chunks, 1024), jnp.int32),
            pltpu.VMEM((1024, D), jnp.float32),
            pltpu.VMEM((8, 2, 8, 128), jnp.float32),
            pltpu.SemaphoreType.DMA,
        ],
    )
    def gather(x_hbm, table_hbm, out5_hbm, x2v, idxT, rows_v, och, gsem):
        wid = lax.axis_index("s") * 2 + lax.axis_index("c")
        pltpu.sync_copy(x_hbm.at[pl.ds(wid * 128, 128), :], x2v)

        def build(rg, carry):
            for s in range(8):
                for lg in range(8):
                    v = plsc.load_gather(
                        x2v,
                        [lg * 16 + _iota16(),
                         jnp.full((16,), 8 * rg + s, jnp.int32)])
                    idxT[rg, pl.ds(s * 128 + lg * 16, 16)] = v
            return carry

        lax.fori_loop(0, n_chunks, build, 0)

        def chunk(rg, carry):
            pltpu.async_copy(table_hbm.at[idxT.at[rg]], rows_v, gsem).wait()
            for s in range(8):
                for g in range(2):
                    for s2 in range(8):
                        for lg in range(8):
                            v = plsc.load_gather(
                                rows_v,
                                [s * 128 + lg * 16 + _iota16(),
                                 jnp.full((16,), 8 * g + s2, jnp.int32)])
                            och[s, g, s2, pl.ds(lg * 16, 16)] = v
            pltpu.sync_copy(och, out5_hbm.at[pl.ds(8 * rg, 8), :, wid])
            return carry

        lax.fori_loop(0, n_chunks, chunk, 0)

    return gather


def kernel(x, table):
    V, D = table.shape
    B, J = x.shape
    out5 = _make_gather_native(B, J, V, D)(x.astype(jnp.int32), table)
    return jnp.transpose(out5, (2, 4, 0, 1, 3)).reshape(B, J, D)


# speedup vs baseline: 1.6012x; 1.1577x over previous
import functools

import jax
import jax.numpy as jnp
from jax import lax
from jax.experimental import pallas as pl
from jax.experimental.pallas import tpu as pltpu
from jax.experimental.pallas import tpu_sc as plsc


def _iota16():
    return lax.iota(jnp.int32, 16)


def _make_gather_native(B4096, J200, V, D):
    mesh = plsc.VectorSubcoreMesh(core_axis_name="c", subcore_axis_name="s")
    n_chunks = J200 // 8

    @functools.partial(
        pl.kernel,
        mesh=mesh,
        compiler_params=pltpu.CompilerParams(
            use_tc_tiling_on_sc=False, needs_layout_passes=False),
        out_type=jax.ShapeDtypeStruct((J200, 2, 32, 8, 128), jnp.float32),
        scratch_types=[
            pltpu.VMEM((128, J200), jnp.int32),
            pltpu.VMEM((n_chunks, 1024), jnp.int32),
            pltpu.VMEM((1024, D), jnp.float32),
            pltpu.VMEM((1024, D), jnp.float32),
            pltpu.VMEM((8, 2, 8, 128), jnp.float32),
            pltpu.SemaphoreType.DMA,
            pltpu.SemaphoreType.DMA,
        ],
    )
    def gather(x_hbm, table_hbm, out5_hbm, x2v, idxT, rows_v, rows_b, och,
               gsem, gsem_b):
        wid = lax.axis_index("s") * 2 + lax.axis_index("c")
        pltpu.sync_copy(x_hbm.at[pl.ds(wid * 128, 128), :], x2v)

        def build(rg, carry):
            for s in range(8):
                for lg in range(8):
                    v = plsc.load_gather(
                        x2v,
                        [lg * 16 + _iota16(),
                         jnp.full((16,), 8 * rg + s, jnp.int32)])
                    idxT[rg, pl.ds(s * 128 + lg * 16, 16)] = v
            return carry

        lax.fori_loop(0, n_chunks, build, 0)

        def transpose_write(rg, rv):
            for s in range(8):
                for g in range(2):
                    for s2 in range(8):
                        vs = [plsc.load_gather(
                                  rv,
                                  [s * 128 + lg * 16 + _iota16(),
                                   jnp.full((16,), 8 * g + s2, jnp.int32)])
                              for lg in range(8)]
                        for lg in range(8):
                            och[s, g, s2, pl.ds(lg * 16, 16)] = vs[lg]
            pltpu.sync_copy(och, out5_hbm.at[pl.ds(8 * rg, 8), :, wid])

        def fire(rg, rv, sem):
            return pltpu.async_copy(table_hbm.at[idxT.at[rg]], rv, sem)

        def drain(rv, sem):
            pltpu.make_async_copy(table_hbm.at[idxT.at[0]], rv, sem).wait()

        fire(0, rows_v, gsem)
        def pair(i, carry):
            fire(2 * i + 1, rows_b, gsem_b)
            drain(rows_v, gsem)
            transpose_write(2 * i, rows_v)
            fire(2 * i + 2, rows_v, gsem)
            drain(rows_b, gsem_b)
            transpose_write(2 * i + 1, rows_b)
            return carry

        lax.fori_loop(0, (n_chunks - 1) // 2, pair, 0)
        drain(rows_v, gsem)
        transpose_write(n_chunks - 1, rows_v)

    return gather


def kernel(x, table):
    V, D = table.shape
    B, J = x.shape
    out5 = _make_gather_native(B, J, V, D)(x.astype(jnp.int32), table)
    return jnp.transpose(out5, (2, 4, 0, 1, 3)).reshape(B, J, D)
